# baseline (speedup 1.0000x reference)
import functools

import jax
import jax.numpy as jnp
from jax import lax
from jax.experimental import pallas as pl
from jax.experimental.pallas import tpu as pltpu

_LANES = 128


def _stats_kernel(feat_ref, labc_ref, sumf_ref, sumsq_ref, cnt_ref):
    s = pl.program_id(1)

    @pl.when(s == 0)
    def _():
        sumf_ref[...] = jnp.zeros_like(sumf_ref)
        sumsq_ref[...] = jnp.zeros_like(sumsq_ref)
        cnt_ref[...] = jnp.zeros_like(cnt_ref)

    f = feat_ref[0]
    labc = labc_ref[...]
    t = labc.shape[0]
    oh = (lax.broadcasted_iota(jnp.int32, (t, _LANES), 1) == labc)
    oh = oh.astype(jnp.float32)
    sumf_ref[0] += jnp.dot(f, oh, preferred_element_type=jnp.float32)
    sumsq_ref[0] += jnp.dot(f * f, oh, preferred_element_type=jnp.float32)
    cnt_ref[0] += jnp.sum(oh, axis=0, keepdims=True)


def _aug_kernel(sumf_ref, sumsq_ref, cnt_ref, w_ref, wt_ref, ratio_ref,
                y_ref, lab_ref, out_ref, st_ref, *, num_classes, n_par):
    s = pl.program_id(1)

    @pl.when(s == 0)
    def _():
        sumf = sumf_ref[0]
        sumsq = sumsq_ref[0]
        cnt = cnt_ref[0]
        for p in range(1, n_par):
            sumf = sumf + sumf_ref[p]
            sumsq = sumsq + sumsq_ref[p]
            cnt = cnt + cnt_ref[p]
        n = jnp.maximum(cnt, 1.0)
        ave = sumf / n
        cv = jnp.maximum(sumsq / n - ave * ave, 0.0)
        col = lax.broadcasted_iota(jnp.int32, cv.shape, 1)
        cv = jnp.where(col < num_classes, cv, 0.0)
        w = w_ref[...]
        wt = wt_ref[...]
        t1 = jnp.dot(w * w, cv, preferred_element_type=jnp.float32)
        t2 = jnp.dot(w, wt * cv, preferred_element_type=jnp.float32)
        t3 = jnp.sum(wt * wt * cv, axis=0, keepdims=True)
        st_ref[...] = ratio_ref[0, 0] * (t1 - 2.0 * t2 + t3)

    lab = lab_ref[...]
    ts = lab.shape[1]
    oh = (lax.broadcasted_iota(jnp.int32, (_LANES, ts), 0) == lab)
    oh = oh.astype(jnp.float32)
    sig = jnp.dot(st_ref[...], oh, preferred_element_type=jnp.float32)
    out_ref[0] = y_ref[0] + 0.5 * sig


def kernel(features, fc_weight_conv, y, target_x, ratio):
    N, A, H, W = features.shape
    C = fc_weight_conv.shape[0]
    HW = H * W

    _, h_in, w_in = target_x.shape
    hi = jnp.floor(jnp.arange(H) * (h_in / H)).astype(jnp.int32)
    wi = jnp.floor(jnp.arange(W) * (w_in / W)).astype(jnp.int32)
    lab = target_x.astype(jnp.float32)[:, hi, :][:, :, wi].astype(jnp.int32)
    lab_flat = lab.reshape(N * HW)
    labc = jnp.where(lab_flat == 255, C, lab_flat).reshape(N * HW, 1)
    labrow = lab_flat.reshape(1, N * HW)

    feat = features.reshape(N, A, HW)
    n_par = 2
    n_inner = N // n_par

    sumf, sumsq, cnt = pl.pallas_call(
        _stats_kernel,
        out_shape=(
            jax.ShapeDtypeStruct((n_par, A, _LANES), jnp.float32),
            jax.ShapeDtypeStruct((n_par, A, _LANES), jnp.float32),
            jax.ShapeDtypeStruct((n_par, 1, _LANES), jnp.float32),
        ),
        grid=(n_par, n_inner),
        in_specs=[
            pl.BlockSpec((1, A, HW), lambda p, s: (p * n_inner + s, 0, 0)),
            pl.BlockSpec((HW, 1), lambda p, s: (p * n_inner + s, 0)),
        ],
        out_specs=(
            pl.BlockSpec((1, A, _LANES), lambda p, s: (p, 0, 0)),
            pl.BlockSpec((1, A, _LANES), lambda p, s: (p, 0, 0)),
            pl.BlockSpec((1, 1, _LANES), lambda p, s: (p, 0, 0)),
        ),
        compiler_params=pltpu.CompilerParams(
            dimension_semantics=("parallel", "arbitrary")),
    )(feat, labc)

    wm = fc_weight_conv.reshape(C, A)
    wt = jnp.zeros((A, _LANES), jnp.float32).at[:, :C].set(wm.T)
    ratio_arr = jnp.asarray(ratio, jnp.float32).reshape(1, 1)
    y3 = y.reshape(N, C, HW).astype(jnp.float32)

    aug = pl.pallas_call(
        functools.partial(_aug_kernel, num_classes=C, n_par=n_par),
        out_shape=jax.ShapeDtypeStruct((N, C, HW), jnp.float32),
        grid=(n_par, n_inner),
        in_specs=[
            pl.BlockSpec((n_par, A, _LANES), lambda p, s: (0, 0, 0)),
            pl.BlockSpec((n_par, A, _LANES), lambda p, s: (0, 0, 0)),
            pl.BlockSpec((n_par, 1, _LANES), lambda p, s: (0, 0, 0)),
            pl.BlockSpec((C, A), lambda p, s: (0, 0)),
            pl.BlockSpec((A, _LANES), lambda p, s: (0, 0)),
            pl.BlockSpec((1, 1), lambda p, s: (0, 0)),
            pl.BlockSpec((1, C, HW), lambda p, s: (p * n_inner + s, 0, 0)),
            pl.BlockSpec((1, HW), lambda p, s: (0, p * n_inner + s)),
        ],
        out_specs=pl.BlockSpec((1, C, HW), lambda p, s: (p * n_inner + s, 0, 0)),
        scratch_shapes=[pltpu.VMEM((C, _LANES), jnp.float32)],
        compiler_params=pltpu.CompilerParams(
            dimension_semantics=("parallel", "arbitrary")),
    )(sumf, sumsq, cnt, wm, wt, ratio_arr, y3, labrow)

    return aug.reshape(N, C, H, W)
